# baseline (speedup 1.0000x reference)
import functools

import jax
import jax.numpy as jnp
from jax import lax
from jax.experimental import pallas as pl
from jax.experimental.pallas import tpu as pltpu
from jax.experimental.pallas import tpu_sc as plsc

S = 5000
P = 4096
N = 50000
SBLK = 512
SPAD = 5120
NBLK = SPAD // SBLK
NROW = 392
NPAD_TOT = NROW * 128
NPAD = NPAD_TOT - N
KSEL = 2500
INT_MIN = -2147483648
INT_MAX = 2147483647


NW = 32
CHUNK = NPAD_TOT // NW
NV = CHUNK // 16


def _sc_axis_body(vx_hbm, vy_hbm, vz_hbm, sx_hbm, sy_hbm, sz_hbm, vb_hbm,
                  out_hbm, bvx, bvy, bvz, bsx, bsy, bsz, vbv, accv):
    wid = lax.axis_index("s") * 2 + lax.axis_index("c")
    base = wid * CHUNK
    pltpu.sync_copy(vx_hbm.at[pl.ds(base, CHUNK)], bvx)
    pltpu.sync_copy(vy_hbm.at[pl.ds(base, CHUNK)], bvy)
    pltpu.sync_copy(vz_hbm.at[pl.ds(base, CHUNK)], bvz)
    pltpu.sync_copy(sx_hbm.at[pl.ds(base, CHUNK)], bsx)
    pltpu.sync_copy(sy_hbm.at[pl.ds(base, CHUNK)], bsy)
    pltpu.sync_copy(sz_hbm.at[pl.ds(base, CHUNK)], bsz)
    pltpu.sync_copy(vb_hbm, vbv)
    v0v = vbv[pl.ds(0, 16)]
    v1v = vbv[pl.ds(16, 16)]
    v2v = vbv[pl.ds(32, 16)]

    def step(j, acc):
        s = pl.ds(j * 16, 16)
        dx = bvx[s] - bsx[s]
        dy = bvy[s] - bsy[s]
        dz = bvz[s] - bsz[s]
        d = dx * v0v + dy * v1v + dz * v2v
        return acc + d * d

    accv[...] = lax.fori_loop(0, NV, step, jnp.zeros((16,), jnp.float32))
    pltpu.sync_copy(accv, out_hbm.at[pl.ds(wid * 16, 16)])


_sc_axis = functools.partial(
    pl.kernel,
    mesh=plsc.VectorSubcoreMesh(core_axis_name="c", subcore_axis_name="s"),
    out_type=jax.ShapeDtypeStruct((NW * 16,), jnp.float32),
    scratch_types=[pltpu.VMEM((CHUNK,), jnp.float32)] * 6
    + [pltpu.VMEM((48,), jnp.float32), pltpu.VMEM((16,), jnp.float32)],
)(_sc_axis_body)


def _regs_compute(vx, vy, vz, sx, sy, sz, v0, v1, v2):
    dx = vx - sx
    dy = vy - sy
    dz = vz - sz
    d = dx * v0 + dy * v1 + dz * v2
    axis_reg = jnp.sum(d * d)

    ps = sx * v0 + sy * v1 + sz * v2
    b = lax.bitcast_convert_type(ps, jnp.int32)
    keys = jnp.where(b >= 0, b, jnp.int32(INT_MIN) - b)
    gi = (lax.broadcasted_iota(jnp.int32, (NROW, 128), 0) * 128
          + lax.broadcasted_iota(jnp.int32, (NROW, 128), 1))
    keys = jnp.where(gi < N, keys, jnp.int32(INT_MAX))
    keys2 = ~keys
    k_lo = jnp.int32(KSEL)
    k_hi = jnp.int32(KSEL + NPAD)

    def vstep(t, pq):
        p, q = pq
        shift = jnp.int32(1) << (31 - t)
        candp = p + shift
        candq = q + shift
        cp = jnp.sum(jnp.where(keys < candp, jnp.int32(1), jnp.int32(0)))
        cq = jnp.sum(jnp.where(keys2 < candq, jnp.int32(1), jnp.int32(0)))
        return (jnp.where(cp < k_lo, candp, p),
                jnp.where(cq < k_hi, candq, q))
    t_lo, t2 = lax.fori_loop(
        0, 32, vstep, (jnp.int32(INT_MIN), jnp.int32(INT_MIN)))

    n_lo = k_lo - jnp.sum(jnp.where(keys < t_lo, jnp.int32(1), jnp.int32(0)))
    n_hi = k_hi - jnp.sum(jnp.where(keys2 < t2, jnp.int32(1), jnp.int32(0)))
    ties_lo = keys == t_lo
    ties_hi = keys2 == t2
    rix = jnp.int32(NPAD_TOT - 1) - gi

    def istep(t, pq):
        p, q = pq
        shift = jnp.int32(1) << (16 - t)
        candp = p + shift
        candq = q + shift
        cp = jnp.sum(jnp.where(ties_lo & (gi < candp),
                               jnp.int32(1), jnp.int32(0)))
        cq = jnp.sum(jnp.where(ties_hi & (rix < candq),
                               jnp.int32(1), jnp.int32(0)))
        return (jnp.where(cp < n_lo, candp, p),
                jnp.where(cq < n_hi, candq, q))
    m_lo, m_hi = lax.fori_loop(0, 17, istep, (jnp.int32(0), jnp.int32(0)))

    mask_lo = (keys < t_lo) | (ties_lo & (gi <= m_lo))
    mask_hi = (keys2 < t2) | (ties_hi & (rix <= m_hi))

    d2v = dx * dx + dy * dy + dz * dz
    br = jnp.sum(d2v * (mask_lo.astype(jnp.float32)
                        + mask_hi.astype(jnp.float32)))
    return axis_reg, br


def _body(sp_ref, snm_ref, snm8_ref, ept_ref, anmt_ref, acat_ref, v_ref,
          vx_ref, vy_ref, vz_ref, sx_ref, sy_ref, sz_ref,
          out_ref, rmin, rpay, raug, acc):
    i = pl.program_id(0)
    v0 = v_ref[0, 0]
    v1 = v_ref[0, 1]
    v2 = v_ref[0, 2]

    e0 = ept_ref[0:1, :]
    e1 = ept_ref[1:2, :]
    e2 = ept_ref[2:3, :]
    c0 = jnp.mean(e0)
    c1 = jnp.mean(e1)
    c2 = jnp.mean(e2)
    p2 = (e0 - c0) * v0 + (e1 - c1) * v1 + (e2 - c2) * v2
    mn = jnp.min(p2)
    mx = jnp.max(p2)

    @pl.when(i == 0)
    def _():
        en = e0 * e0 + e1 * e1 + e2 * e2
        raug[...] = jnp.concatenate(
            [-2.0 * e0, -2.0 * e1, -2.0 * e2,
             jnp.ones((1, P), jnp.float32), en,
             jnp.ones((1, P), jnp.float32),
             jnp.zeros((2, P), jnp.float32)], axis=0)
        axis_reg, br = _regs_compute(
            vx_ref[...], vy_ref[...], vz_ref[...],
            sx_ref[...], sy_ref[...], sz_ref[...], v0, v1, v2)
        out_ref[0, 2] = axis_reg
        out_ref[0, 3] = br

    x0 = sp_ref[:, 0:1]
    x1 = sp_ref[:, 1:2]
    x2 = sp_ref[:, 2:3]
    proj1 = (x0 - c0) * v0 + (x1 - c1) * v1 + (x2 - c2) * v2
    rows = i * SBLK + lax.broadcasted_iota(jnp.int32, (SBLK, 1), 0)
    valid = rows < S
    maskf = ((proj1 >= mn) & (proj1 <= mx) & valid).astype(jnp.float32)
    pen = (1.0 - maskf) * 1e10 + jnp.where(valid, 0.0, 1e20)

    sn = x0 * x0 + x1 * x1 + x2 * x2
    lhs = jnp.concatenate(
        [x0, x1, x2, sn, jnp.ones((SBLK, 1), jnp.float32), pen,
         jnp.zeros((SBLK, 2), jnp.float32)], axis=1)
    dm = jnp.maximum(
        jnp.dot(lhs, raug[...], preferred_element_type=jnp.float32), 0.0)

    minx = jnp.min(dm, axis=1, keepdims=True)
    ohr = (dm == minx).astype(jnp.float32)
    g = jnp.dot(ohr, acat_ref[...], preferred_element_type=jnp.float32)
    g0 = g[:, 0:1]
    g1 = g[:, 1:2]
    g2 = g[:, 2:3]
    n0 = snm_ref[:, 0:1]
    n1 = snm_ref[:, 1:2]
    n2 = snm_ref[:, 2:3]
    sn2 = n0 * n0 + n1 * n1 + n2 * n2
    an2_sel = g0 * g0 + g1 * g1 + g2 * g2
    ndot_r = n0 * g0 + n1 * g1 + n2 * g2
    cosxy = ndot_r / (jnp.sqrt(sn2) * jnp.sqrt(an2_sel) + 1e-8)

    s_lossx = jnp.sum(minx * maskf)
    s_denom = jnp.sum(maskf)
    s_lnx = jnp.sum((1.0 - jnp.abs(cosxy)) * maskf)

    @pl.when(i == 0)
    def _():
        acc[0] = s_lossx
        acc[1] = s_denom
        acc[2] = s_lnx

    @pl.when(i > 0)
    def _():
        acc[0] = acc[0] + s_lossx
        acc[1] = acc[1] + s_denom
        acc[2] = acc[2] + s_lnx

    m0 = jnp.min(dm, axis=0, keepdims=True)
    rmin_old = jnp.where(i == 0, jnp.float32(3e38), rmin[...])
    better = m0 < rmin_old
    rmin[...] = jnp.where(better, m0, rmin_old)
    ohcT = jnp.where((dm == m0) & better, 1.0, 0.0)
    gc = jnp.dot(snm8_ref[...], ohcT, preferred_element_type=jnp.float32)
    rpay[...] = jnp.where(better, gc, rpay[...])

    @pl.when(i == NBLK - 1)
    def _():
        lossy = jnp.sum(rmin[...]) / P
        a0 = anmt_ref[0:1, :]
        a1 = anmt_ref[1:2, :]
        a2 = anmt_ref[2:3, :]
        an2 = a0 * a0 + a1 * a1 + a2 * a2
        pay = rpay[...]
        p0 = pay[0:1, :]
        p1 = pay[1:2, :]
        p2s = pay[2:3, :]
        sn2_sel = p0 * p0 + p1 * p1 + p2s * p2s
        ndot_c = a0 * p0 + a1 * p1 + a2 * p2s
        cosyx = ndot_c / (jnp.sqrt(an2) * jnp.sqrt(sn2_sel) + 1e-8)
        lny = jnp.sum(1.0 - jnp.abs(cosyx)) / P
        denom = jnp.maximum(acc[1], 1.0)
        out_ref[0, 0] = acc[0] / denom + lossy
        out_ref[0, 1] = acc[2] / denom + lny


def _pad_rows(x, n):
    return jnp.concatenate(
        [x, jnp.zeros((n - x.shape[0], x.shape[1]), x.dtype)], axis=0)


@jax.jit
def kernel(sample_pcd, sample_norm, echo_pcd, aligned_pcd_norm, pcd_vector,
           verts, static_verts):
    sp = _pad_rows(sample_pcd, SPAD)
    snm = _pad_rows(sample_norm, SPAD)
    snm8 = _pad_rows(snm.T, 8)
    ept = echo_pcd.T
    anmt = aligned_pcd_norm.T
    acat = jnp.concatenate(
        [aligned_pcd_norm, jnp.zeros((P, 5), jnp.float32)], axis=1)
    vt = jnp.concatenate(
        [verts.T, jnp.zeros((3, NPAD_TOT - N), verts.dtype)], axis=1)
    st = jnp.concatenate(
        [static_verts.T, jnp.zeros((3, NPAD_TOT - N), verts.dtype)], axis=1)
    vr = vt.reshape(3, NROW, 128)
    sr = st.reshape(3, NROW, 128)

    vb = jnp.repeat(pcd_vector[0], 16)
    scax = _sc_axis(vt[0], vt[1], vt[2], st[0], st[1], st[2], vb)

    full = lambda i: (0, 0)
    out = pl.pallas_call(
        _body,
        grid=(NBLK,),
        in_specs=[
            pl.BlockSpec((SBLK, 3), lambda i: (i, 0)),
            pl.BlockSpec((SBLK, 3), lambda i: (i, 0)),
            pl.BlockSpec((8, SBLK), lambda i: (0, i)),
            pl.BlockSpec((3, P), full),
            pl.BlockSpec((3, P), full),
            pl.BlockSpec((P, 8), full),
            pl.BlockSpec(memory_space=pltpu.SMEM),
            pl.BlockSpec((NROW, 128), full),
            pl.BlockSpec((NROW, 128), full),
            pl.BlockSpec((NROW, 128), full),
            pl.BlockSpec((NROW, 128), full),
            pl.BlockSpec((NROW, 128), full),
            pl.BlockSpec((NROW, 128), full),
        ],
        out_specs=pl.BlockSpec(memory_space=pltpu.SMEM),
        out_shape=jax.ShapeDtypeStruct((1, 4), jnp.float32),
        scratch_shapes=[
            pltpu.VMEM((1, P), jnp.float32),
            pltpu.VMEM((8, P), jnp.float32),
            pltpu.VMEM((8, P), jnp.float32),
            pltpu.SMEM((8,), jnp.float32),
        ],
        compiler_params=pltpu.CompilerParams(
            dimension_semantics=("arbitrary",)),
    )(sp, snm, snm8, ept, anmt, acat, pcd_vector,
      vr[0], vr[1], vr[2], sr[0], sr[1], sr[2])

    return jnp.stack([out[0, 0], out[0, 1], jnp.sum(scax), out[0, 3]])


# speedup vs baseline: 1.3864x; 1.3864x over previous
import jax
import jax.numpy as jnp
from jax import lax
from jax.experimental import pallas as pl
from jax.experimental.pallas import tpu as pltpu

S = 5000
P = 4096
N = 50000
SBLK = 512
SPAD = 5120
NBLK = SPAD // SBLK
NROW = 392
NPAD_TOT = NROW * 128
NPAD = NPAD_TOT - N
KSEL = 2500
INT_MIN = -2147483648
INT_MAX = 2147483647


def _regs_compute(vx, vy, vz, sx, sy, sz, v0, v1, v2):
    dx = vx - sx
    dy = vy - sy
    dz = vz - sz
    d = dx * v0 + dy * v1 + dz * v2
    axis_reg = jnp.sum(d * d)

    ps = sx * v0 + sy * v1 + sz * v2
    b = lax.bitcast_convert_type(ps, jnp.int32)
    keys = jnp.where(b >= 0, b, jnp.int32(INT_MIN) - b)
    gi = (lax.broadcasted_iota(jnp.int32, (NROW, 128), 0) * 128
          + lax.broadcasted_iota(jnp.int32, (NROW, 128), 1))
    keys = jnp.where(gi < N, keys, jnp.int32(INT_MAX))
    keys2 = ~keys
    k_lo = jnp.int32(KSEL)
    k_hi = jnp.int32(KSEL + NPAD)

    def vstep(t, pq):
        p, q = pq
        shift = jnp.int32(1) << (31 - t)
        candp = p + shift
        candq = q + shift
        cp = jnp.sum(jnp.where(keys < candp, jnp.int32(1), jnp.int32(0)))
        cq = jnp.sum(jnp.where(keys2 < candq, jnp.int32(1), jnp.int32(0)))
        return (jnp.where(cp < k_lo, candp, p),
                jnp.where(cq < k_hi, candq, q))
    t_lo, t2 = lax.fori_loop(
        0, 32, vstep, (jnp.int32(INT_MIN), jnp.int32(INT_MIN)))

    n_lo = k_lo - jnp.sum(jnp.where(keys < t_lo, jnp.int32(1), jnp.int32(0)))
    n_hi = k_hi - jnp.sum(jnp.where(keys2 < t2, jnp.int32(1), jnp.int32(0)))
    ties_lo = keys == t_lo
    ties_hi = keys2 == t2
    rix = jnp.int32(NPAD_TOT - 1) - gi

    def istep(t, pq):
        p, q = pq
        shift = jnp.int32(1) << (16 - t)
        candp = p + shift
        candq = q + shift
        cp = jnp.sum(jnp.where(ties_lo & (gi < candp),
                               jnp.int32(1), jnp.int32(0)))
        cq = jnp.sum(jnp.where(ties_hi & (rix < candq),
                               jnp.int32(1), jnp.int32(0)))
        return (jnp.where(cp < n_lo, candp, p),
                jnp.where(cq < n_hi, candq, q))
    m_lo, m_hi = lax.fori_loop(0, 17, istep, (jnp.int32(0), jnp.int32(0)))

    mask_lo = (keys < t_lo) | (ties_lo & (gi <= m_lo))
    mask_hi = (keys2 < t2) | (ties_hi & (rix <= m_hi))

    d2v = dx * dx + dy * dy + dz * dz
    br = jnp.sum(d2v * (mask_lo.astype(jnp.float32)
                        + mask_hi.astype(jnp.float32)))
    return axis_reg, br


def _body(sp_ref, snm_ref, snm8_ref, ept_ref, anmt_ref, acat_ref, v_ref,
          vx_ref, vy_ref, vz_ref, sx_ref, sy_ref, sz_ref,
          out_ref, rmin, rpay, raug, acc):
    i = pl.program_id(0)
    v0 = v_ref[0, 0]
    v1 = v_ref[0, 1]
    v2 = v_ref[0, 2]

    e0 = ept_ref[0:1, :]
    e1 = ept_ref[1:2, :]
    e2 = ept_ref[2:3, :]
    c0 = jnp.mean(e0)
    c1 = jnp.mean(e1)
    c2 = jnp.mean(e2)
    p2 = (e0 - c0) * v0 + (e1 - c1) * v1 + (e2 - c2) * v2
    mn = jnp.min(p2)
    mx = jnp.max(p2)

    @pl.when(i == 0)
    def _():
        en = e0 * e0 + e1 * e1 + e2 * e2
        raug[...] = jnp.concatenate(
            [-2.0 * e0, -2.0 * e1, -2.0 * e2,
             jnp.ones((1, P), jnp.float32), en,
             jnp.ones((1, P), jnp.float32),
             jnp.zeros((2, P), jnp.float32)], axis=0)
        axis_reg, br = _regs_compute(
            vx_ref[...], vy_ref[...], vz_ref[...],
            sx_ref[...], sy_ref[...], sz_ref[...], v0, v1, v2)
        out_ref[0, 2] = axis_reg
        out_ref[0, 3] = br

    x0 = sp_ref[:, 0:1]
    x1 = sp_ref[:, 1:2]
    x2 = sp_ref[:, 2:3]
    proj1 = (x0 - c0) * v0 + (x1 - c1) * v1 + (x2 - c2) * v2
    rows = i * SBLK + lax.broadcasted_iota(jnp.int32, (SBLK, 1), 0)
    valid = rows < S
    maskf = ((proj1 >= mn) & (proj1 <= mx) & valid).astype(jnp.float32)
    pen = (1.0 - maskf) * 1e10 + jnp.where(valid, 0.0, 1e20)

    sn = x0 * x0 + x1 * x1 + x2 * x2
    lhs = jnp.concatenate(
        [x0, x1, x2, sn, jnp.ones((SBLK, 1), jnp.float32), pen,
         jnp.zeros((SBLK, 2), jnp.float32)], axis=1)
    dm = jnp.maximum(
        jnp.dot(lhs, raug[...], preferred_element_type=jnp.float32), 0.0)

    minx = jnp.min(dm, axis=1, keepdims=True)
    ohr = (dm == minx).astype(jnp.float32)
    g = jnp.dot(ohr, acat_ref[...], preferred_element_type=jnp.float32)
    g0 = g[:, 0:1]
    g1 = g[:, 1:2]
    g2 = g[:, 2:3]
    n0 = snm_ref[:, 0:1]
    n1 = snm_ref[:, 1:2]
    n2 = snm_ref[:, 2:3]
    sn2 = n0 * n0 + n1 * n1 + n2 * n2
    an2_sel = g0 * g0 + g1 * g1 + g2 * g2
    ndot_r = n0 * g0 + n1 * g1 + n2 * g2
    cosxy = ndot_r / (jnp.sqrt(sn2) * jnp.sqrt(an2_sel) + 1e-8)

    s_lossx = jnp.sum(minx * maskf)
    s_denom = jnp.sum(maskf)
    s_lnx = jnp.sum((1.0 - jnp.abs(cosxy)) * maskf)

    @pl.when(i == 0)
    def _():
        acc[0] = s_lossx
        acc[1] = s_denom
        acc[2] = s_lnx

    @pl.when(i > 0)
    def _():
        acc[0] = acc[0] + s_lossx
        acc[1] = acc[1] + s_denom
        acc[2] = acc[2] + s_lnx

    m0 = jnp.min(dm, axis=0, keepdims=True)
    rmin_old = jnp.where(i == 0, jnp.float32(3e38), rmin[...])
    better = m0 < rmin_old
    rmin[...] = jnp.where(better, m0, rmin_old)
    ohcT = jnp.where(dm == m0, 1.0, 0.0)
    gc = jnp.dot(snm8_ref[...], ohcT, preferred_element_type=jnp.float32)
    rpay[...] = jnp.where(better, gc, rpay[...])

    @pl.when(i == NBLK - 1)
    def _():
        lossy = jnp.sum(rmin[...]) / P
        a0 = anmt_ref[0:1, :]
        a1 = anmt_ref[1:2, :]
        a2 = anmt_ref[2:3, :]
        an2 = a0 * a0 + a1 * a1 + a2 * a2
        pay = rpay[...]
        p0 = pay[0:1, :]
        p1 = pay[1:2, :]
        p2s = pay[2:3, :]
        sn2_sel = p0 * p0 + p1 * p1 + p2s * p2s
        ndot_c = a0 * p0 + a1 * p1 + a2 * p2s
        cosyx = ndot_c / (jnp.sqrt(an2) * jnp.sqrt(sn2_sel) + 1e-8)
        lny = jnp.sum(1.0 - jnp.abs(cosyx)) / P
        denom = jnp.maximum(acc[1], 1.0)
        out_ref[0, 0] = acc[0] / denom + lossy
        out_ref[0, 1] = acc[2] / denom + lny


def _pad_rows(x, n):
    return jnp.concatenate(
        [x, jnp.zeros((n - x.shape[0], x.shape[1]), x.dtype)], axis=0)


@jax.jit
def kernel(sample_pcd, sample_norm, echo_pcd, aligned_pcd_norm, pcd_vector,
           verts, static_verts):
    sp = _pad_rows(sample_pcd, SPAD)
    snm = _pad_rows(sample_norm, SPAD)
    snm8 = _pad_rows(snm.T, 8)
    ept = echo_pcd.T
    anmt = aligned_pcd_norm.T
    acat = jnp.concatenate(
        [aligned_pcd_norm, jnp.zeros((P, 5), jnp.float32)], axis=1)
    vt = jnp.concatenate(
        [verts.T, jnp.zeros((3, NPAD_TOT - N), verts.dtype)], axis=1)
    st = jnp.concatenate(
        [static_verts.T, jnp.zeros((3, NPAD_TOT - N), verts.dtype)], axis=1)
    vr = vt.reshape(3, NROW, 128)
    sr = st.reshape(3, NROW, 128)

    full = lambda i: (0, 0)
    out = pl.pallas_call(
        _body,
        grid=(NBLK,),
        in_specs=[
            pl.BlockSpec((SBLK, 3), lambda i: (i, 0)),
            pl.BlockSpec((SBLK, 3), lambda i: (i, 0)),
            pl.BlockSpec((8, SBLK), lambda i: (0, i)),
            pl.BlockSpec((3, P), full),
            pl.BlockSpec((3, P), full),
            pl.BlockSpec((P, 8), full),
            pl.BlockSpec(memory_space=pltpu.SMEM),
            pl.BlockSpec((NROW, 128), full),
            pl.BlockSpec((NROW, 128), full),
            pl.BlockSpec((NROW, 128), full),
            pl.BlockSpec((NROW, 128), full),
            pl.BlockSpec((NROW, 128), full),
            pl.BlockSpec((NROW, 128), full),
        ],
        out_specs=pl.BlockSpec(memory_space=pltpu.SMEM),
        out_shape=jax.ShapeDtypeStruct((1, 4), jnp.float32),
        scratch_shapes=[
            pltpu.VMEM((1, P), jnp.float32),
            pltpu.VMEM((8, P), jnp.float32),
            pltpu.VMEM((8, P), jnp.float32),
            pltpu.SMEM((8,), jnp.float32),
        ],
        compiler_params=pltpu.CompilerParams(
            dimension_semantics=("arbitrary",)),
    )(sp, snm, snm8, ept, anmt, acat, pcd_vector,
      vr[0], vr[1], vr[2], sr[0], sr[1], sr[2])

    return out[0]


# speedup vs baseline: 1.4126x; 1.0189x over previous
import jax
import jax.numpy as jnp
from jax import lax
from jax.experimental import pallas as pl
from jax.experimental.pallas import tpu as pltpu

S = 5000
P = 4096
N = 50000
SBLK = 640
SPAD = 5120
NBLK = SPAD // SBLK
NROW = 392
NPAD_TOT = NROW * 128
NPAD = NPAD_TOT - N
KSEL = 2500
INT_MIN = -2147483648
INT_MAX = 2147483647


def _regs_compute(vx, vy, vz, sx, sy, sz, v0, v1, v2):
    dx = vx - sx
    dy = vy - sy
    dz = vz - sz
    d = dx * v0 + dy * v1 + dz * v2
    axis_reg = jnp.sum(d * d)

    ps = sx * v0 + sy * v1 + sz * v2
    b = lax.bitcast_convert_type(ps, jnp.int32)
    keys = jnp.where(b >= 0, b, jnp.int32(INT_MIN) - b)
    gi = (lax.broadcasted_iota(jnp.int32, (NROW, 128), 0) * 128
          + lax.broadcasted_iota(jnp.int32, (NROW, 128), 1))
    keys = jnp.where(gi < N, keys, jnp.int32(INT_MAX))
    keys2 = ~keys
    k_lo = jnp.int32(KSEL)
    k_hi = jnp.int32(KSEL + NPAD)

    def vstep(t, pq):
        p, q = pq
        shift = jnp.int32(1) << (31 - t)
        candp = p + shift
        candq = q + shift
        cp = jnp.sum(jnp.where(keys < candp, jnp.int32(1), jnp.int32(0)))
        cq = jnp.sum(jnp.where(keys2 < candq, jnp.int32(1), jnp.int32(0)))
        return (jnp.where(cp < k_lo, candp, p),
                jnp.where(cq < k_hi, candq, q))
    t_lo, t2 = lax.fori_loop(
        0, 32, vstep, (jnp.int32(INT_MIN), jnp.int32(INT_MIN)))

    n_lo = k_lo - jnp.sum(jnp.where(keys < t_lo, jnp.int32(1), jnp.int32(0)))
    n_hi = k_hi - jnp.sum(jnp.where(keys2 < t2, jnp.int32(1), jnp.int32(0)))
    ties_lo = keys == t_lo
    ties_hi = keys2 == t2
    rix = jnp.int32(NPAD_TOT - 1) - gi

    def istep(t, pq):
        p, q = pq
        shift = jnp.int32(1) << (16 - t)
        candp = p + shift
        candq = q + shift
        cp = jnp.sum(jnp.where(ties_lo & (gi < candp),
                               jnp.int32(1), jnp.int32(0)))
        cq = jnp.sum(jnp.where(ties_hi & (rix < candq),
                               jnp.int32(1), jnp.int32(0)))
        return (jnp.where(cp < n_lo, candp, p),
                jnp.where(cq < n_hi, candq, q))
    m_lo, m_hi = lax.fori_loop(0, 17, istep, (jnp.int32(0), jnp.int32(0)))

    mask_lo = (keys < t_lo) | (ties_lo & (gi <= m_lo))
    mask_hi = (keys2 < t2) | (ties_hi & (rix <= m_hi))

    d2v = dx * dx + dy * dy + dz * dz
    br = jnp.sum(d2v * (mask_lo.astype(jnp.float32)
                        + mask_hi.astype(jnp.float32)))
    return axis_reg, br


def _body(sp_ref, snm_ref, snm8_ref, ept_ref, anmt_ref, acat_ref, v_ref,
          vx_ref, vy_ref, vz_ref, sx_ref, sy_ref, sz_ref,
          out_ref, rmin, rpay, raug, acc):
    i = pl.program_id(0)
    v0 = v_ref[0, 0]
    v1 = v_ref[0, 1]
    v2 = v_ref[0, 2]

    e0 = ept_ref[0:1, :]
    e1 = ept_ref[1:2, :]
    e2 = ept_ref[2:3, :]
    c0 = jnp.mean(e0)
    c1 = jnp.mean(e1)
    c2 = jnp.mean(e2)
    p2 = (e0 - c0) * v0 + (e1 - c1) * v1 + (e2 - c2) * v2
    mn = jnp.min(p2)
    mx = jnp.max(p2)

    @pl.when(i == 0)
    def _():
        en = e0 * e0 + e1 * e1 + e2 * e2
        raug[...] = jnp.concatenate(
            [-2.0 * e0, -2.0 * e1, -2.0 * e2,
             jnp.ones((1, P), jnp.float32), en,
             jnp.ones((1, P), jnp.float32),
             jnp.zeros((2, P), jnp.float32)], axis=0)
        axis_reg, br = _regs_compute(
            vx_ref[...], vy_ref[...], vz_ref[...],
            sx_ref[...], sy_ref[...], sz_ref[...], v0, v1, v2)
        out_ref[0, 2] = axis_reg
        out_ref[0, 3] = br

    x0 = sp_ref[:, 0:1]
    x1 = sp_ref[:, 1:2]
    x2 = sp_ref[:, 2:3]
    proj1 = (x0 - c0) * v0 + (x1 - c1) * v1 + (x2 - c2) * v2
    rows = i * SBLK + lax.broadcasted_iota(jnp.int32, (SBLK, 1), 0)
    valid = rows < S
    maskf = ((proj1 >= mn) & (proj1 <= mx) & valid).astype(jnp.float32)
    pen = (1.0 - maskf) * 1e10 + jnp.where(valid, 0.0, 1e20)

    sn = x0 * x0 + x1 * x1 + x2 * x2
    lhs = jnp.concatenate(
        [x0, x1, x2, sn, jnp.ones((SBLK, 1), jnp.float32), pen,
         jnp.zeros((SBLK, 2), jnp.float32)], axis=1)
    dm = jnp.maximum(
        jnp.dot(lhs, raug[...], preferred_element_type=jnp.float32), 0.0)

    minx = jnp.min(dm, axis=1, keepdims=True)
    ohr = (dm == minx).astype(jnp.float32)
    g = jnp.dot(ohr, acat_ref[...], preferred_element_type=jnp.float32)
    g0 = g[:, 0:1]
    g1 = g[:, 1:2]
    g2 = g[:, 2:3]
    n0 = snm_ref[:, 0:1]
    n1 = snm_ref[:, 1:2]
    n2 = snm_ref[:, 2:3]
    sn2 = n0 * n0 + n1 * n1 + n2 * n2
    an2_sel = g0 * g0 + g1 * g1 + g2 * g2
    ndot_r = n0 * g0 + n1 * g1 + n2 * g2
    cosxy = ndot_r / (jnp.sqrt(sn2) * jnp.sqrt(an2_sel) + 1e-8)

    s_lossx = jnp.sum(minx * maskf)
    s_denom = jnp.sum(maskf)
    s_lnx = jnp.sum((1.0 - jnp.abs(cosxy)) * maskf)

    @pl.when(i == 0)
    def _():
        acc[0] = s_lossx
        acc[1] = s_denom
        acc[2] = s_lnx

    @pl.when(i > 0)
    def _():
        acc[0] = acc[0] + s_lossx
        acc[1] = acc[1] + s_denom
        acc[2] = acc[2] + s_lnx

    m0 = jnp.min(dm, axis=0, keepdims=True)
    rmin_old = jnp.where(i == 0, jnp.float32(3e38), rmin[...])
    better = m0 < rmin_old
    rmin[...] = jnp.where(better, m0, rmin_old)
    ohcT = jnp.where(dm == m0, 1.0, 0.0)
    gc = jnp.dot(snm8_ref[...], ohcT, preferred_element_type=jnp.float32)
    rpay[...] = jnp.where(better, gc, rpay[...])

    @pl.when(i == NBLK - 1)
    def _():
        lossy = jnp.sum(rmin[...]) / P
        a0 = anmt_ref[0:1, :]
        a1 = anmt_ref[1:2, :]
        a2 = anmt_ref[2:3, :]
        an2 = a0 * a0 + a1 * a1 + a2 * a2
        pay = rpay[...]
        p0 = pay[0:1, :]
        p1 = pay[1:2, :]
        p2s = pay[2:3, :]
        sn2_sel = p0 * p0 + p1 * p1 + p2s * p2s
        ndot_c = a0 * p0 + a1 * p1 + a2 * p2s
        cosyx = ndot_c / (jnp.sqrt(an2) * jnp.sqrt(sn2_sel) + 1e-8)
        lny = jnp.sum(1.0 - jnp.abs(cosyx)) / P
        denom = jnp.maximum(acc[1], 1.0)
        out_ref[0, 0] = acc[0] / denom + lossy
        out_ref[0, 1] = acc[2] / denom + lny


def _pad_rows(x, n):
    return jnp.concatenate(
        [x, jnp.zeros((n - x.shape[0], x.shape[1]), x.dtype)], axis=0)


@jax.jit
def kernel(sample_pcd, sample_norm, echo_pcd, aligned_pcd_norm, pcd_vector,
           verts, static_verts):
    sp = _pad_rows(sample_pcd, SPAD)
    snm = _pad_rows(sample_norm, SPAD)
    snm8 = _pad_rows(snm.T, 8)
    ept = echo_pcd.T
    anmt = aligned_pcd_norm.T
    acat = jnp.concatenate(
        [aligned_pcd_norm, jnp.zeros((P, 5), jnp.float32)], axis=1)
    vt = jnp.concatenate(
        [verts.T, jnp.zeros((3, NPAD_TOT - N), verts.dtype)], axis=1)
    st = jnp.concatenate(
        [static_verts.T, jnp.zeros((3, NPAD_TOT - N), verts.dtype)], axis=1)
    vr = vt.reshape(3, NROW, 128)
    sr = st.reshape(3, NROW, 128)

    full = lambda i: (0, 0)
    out = pl.pallas_call(
        _body,
        grid=(NBLK,),
        in_specs=[
            pl.BlockSpec((SBLK, 3), lambda i: (i, 0)),
            pl.BlockSpec((SBLK, 3), lambda i: (i, 0)),
            pl.BlockSpec((8, SBLK), lambda i: (0, i)),
            pl.BlockSpec((3, P), full),
            pl.BlockSpec((3, P), full),
            pl.BlockSpec((P, 8), full),
            pl.BlockSpec(memory_space=pltpu.SMEM),
            pl.BlockSpec((NROW, 128), full),
            pl.BlockSpec((NROW, 128), full),
            pl.BlockSpec((NROW, 128), full),
            pl.BlockSpec((NROW, 128), full),
            pl.BlockSpec((NROW, 128), full),
            pl.BlockSpec((NROW, 128), full),
        ],
        out_specs=pl.BlockSpec(memory_space=pltpu.SMEM),
        out_shape=jax.ShapeDtypeStruct((1, 4), jnp.float32),
        scratch_shapes=[
            pltpu.VMEM((1, P), jnp.float32),
            pltpu.VMEM((8, P), jnp.float32),
            pltpu.VMEM((8, P), jnp.float32),
            pltpu.SMEM((8,), jnp.float32),
        ],
        compiler_params=pltpu.CompilerParams(
            dimension_semantics=("arbitrary",)),
    )(sp, snm, snm8, ept, anmt, acat, pcd_vector,
      vr[0], vr[1], vr[2], sr[0], sr[1], sr[2])

    return out[0]


# speedup vs baseline: 1.4604x; 1.0338x over previous
import jax
import jax.numpy as jnp
from jax import lax
from jax.experimental import pallas as pl
from jax.experimental.pallas import tpu as pltpu

S = 5000
P = 4096
N = 50000
SBLK = 1024
SPAD = 5120
NBLK = SPAD // SBLK
NROW = 392
NPAD_TOT = NROW * 128
NPAD = NPAD_TOT - N
KSEL = 2500
INT_MIN = -2147483648
INT_MAX = 2147483647


def _regs_compute(vx, vy, vz, sx, sy, sz, v0, v1, v2):
    dx = vx - sx
    dy = vy - sy
    dz = vz - sz
    d = dx * v0 + dy * v1 + dz * v2
    axis_reg = jnp.sum(d * d)

    ps = sx * v0 + sy * v1 + sz * v2
    b = lax.bitcast_convert_type(ps, jnp.int32)
    keys = jnp.where(b >= 0, b, jnp.int32(INT_MIN) - b)
    gi = (lax.broadcasted_iota(jnp.int32, (NROW, 128), 0) * 128
          + lax.broadcasted_iota(jnp.int32, (NROW, 128), 1))
    keys = jnp.where(gi < N, keys, jnp.int32(INT_MAX))
    keys2 = ~keys
    k_lo = jnp.int32(KSEL)
    k_hi = jnp.int32(KSEL + NPAD)

    def vstep(t, pq):
        p, q = pq
        shift = jnp.int32(1) << (31 - t)
        candp = p + shift
        candq = q + shift
        cp = jnp.sum(jnp.where(keys < candp, jnp.int32(1), jnp.int32(0)))
        cq = jnp.sum(jnp.where(keys2 < candq, jnp.int32(1), jnp.int32(0)))
        return (jnp.where(cp < k_lo, candp, p),
                jnp.where(cq < k_hi, candq, q))
    t_lo, t2 = lax.fori_loop(
        0, 32, vstep, (jnp.int32(INT_MIN), jnp.int32(INT_MIN)))

    n_lo = k_lo - jnp.sum(jnp.where(keys < t_lo, jnp.int32(1), jnp.int32(0)))
    n_hi = k_hi - jnp.sum(jnp.where(keys2 < t2, jnp.int32(1), jnp.int32(0)))
    ties_lo = keys == t_lo
    ties_hi = keys2 == t2
    rix = jnp.int32(NPAD_TOT - 1) - gi

    def istep(t, pq):
        p, q = pq
        shift = jnp.int32(1) << (16 - t)
        candp = p + shift
        candq = q + shift
        cp = jnp.sum(jnp.where(ties_lo & (gi < candp),
                               jnp.int32(1), jnp.int32(0)))
        cq = jnp.sum(jnp.where(ties_hi & (rix < candq),
                               jnp.int32(1), jnp.int32(0)))
        return (jnp.where(cp < n_lo, candp, p),
                jnp.where(cq < n_hi, candq, q))
    m_lo, m_hi = lax.fori_loop(0, 17, istep, (jnp.int32(0), jnp.int32(0)))

    mask_lo = (keys < t_lo) | (ties_lo & (gi <= m_lo))
    mask_hi = (keys2 < t2) | (ties_hi & (rix <= m_hi))

    d2v = dx * dx + dy * dy + dz * dz
    br = jnp.sum(d2v * (mask_lo.astype(jnp.float32)
                        + mask_hi.astype(jnp.float32)))
    return axis_reg, br


def _body(sp_ref, snm_ref, snm8_ref, ept_ref, anmt_ref, acat_ref, v_ref,
          vx_ref, vy_ref, vz_ref, sx_ref, sy_ref, sz_ref,
          out_ref, rmin, rpay, raug, acc):
    i = pl.program_id(0)
    v0 = v_ref[0, 0]
    v1 = v_ref[0, 1]
    v2 = v_ref[0, 2]

    e0 = ept_ref[0:1, :]
    e1 = ept_ref[1:2, :]
    e2 = ept_ref[2:3, :]
    c0 = jnp.mean(e0)
    c1 = jnp.mean(e1)
    c2 = jnp.mean(e2)
    p2 = (e0 - c0) * v0 + (e1 - c1) * v1 + (e2 - c2) * v2
    mn = jnp.min(p2)
    mx = jnp.max(p2)

    @pl.when(i == 0)
    def _():
        en = e0 * e0 + e1 * e1 + e2 * e2
        raug[...] = jnp.concatenate(
            [-2.0 * e0, -2.0 * e1, -2.0 * e2,
             jnp.ones((1, P), jnp.float32), en,
             jnp.ones((1, P), jnp.float32),
             jnp.zeros((2, P), jnp.float32)], axis=0)
        axis_reg, br = _regs_compute(
            vx_ref[...], vy_ref[...], vz_ref[...],
            sx_ref[...], sy_ref[...], sz_ref[...], v0, v1, v2)
        out_ref[0, 2] = axis_reg
        out_ref[0, 3] = br

    x0 = sp_ref[:, 0:1]
    x1 = sp_ref[:, 1:2]
    x2 = sp_ref[:, 2:3]
    proj1 = (x0 - c0) * v0 + (x1 - c1) * v1 + (x2 - c2) * v2
    rows = i * SBLK + lax.broadcasted_iota(jnp.int32, (SBLK, 1), 0)
    valid = rows < S
    maskf = ((proj1 >= mn) & (proj1 <= mx) & valid).astype(jnp.float32)
    pen = (1.0 - maskf) * 1e10 + jnp.where(valid, 0.0, 1e20)

    sn = x0 * x0 + x1 * x1 + x2 * x2
    lhs = jnp.concatenate(
        [x0, x1, x2, sn, jnp.ones((SBLK, 1), jnp.float32), pen,
         jnp.zeros((SBLK, 2), jnp.float32)], axis=1)
    dm = jnp.maximum(
        jnp.dot(lhs, raug[...], preferred_element_type=jnp.float32), 0.0)

    minx = jnp.min(dm, axis=1, keepdims=True)
    ohr = (dm == minx).astype(jnp.float32)
    g = jnp.dot(ohr, acat_ref[...], preferred_element_type=jnp.float32)
    g0 = g[:, 0:1]
    g1 = g[:, 1:2]
    g2 = g[:, 2:3]
    n0 = snm_ref[:, 0:1]
    n1 = snm_ref[:, 1:2]
    n2 = snm_ref[:, 2:3]
    sn2 = n0 * n0 + n1 * n1 + n2 * n2
    an2_sel = g0 * g0 + g1 * g1 + g2 * g2
    ndot_r = n0 * g0 + n1 * g1 + n2 * g2
    cosxy = ndot_r / (jnp.sqrt(sn2) * jnp.sqrt(an2_sel) + 1e-8)

    s_lossx = jnp.sum(minx * maskf)
    s_denom = jnp.sum(maskf)
    s_lnx = jnp.sum((1.0 - jnp.abs(cosxy)) * maskf)

    @pl.when(i == 0)
    def _():
        acc[0] = s_lossx
        acc[1] = s_denom
        acc[2] = s_lnx

    @pl.when(i > 0)
    def _():
        acc[0] = acc[0] + s_lossx
        acc[1] = acc[1] + s_denom
        acc[2] = acc[2] + s_lnx

    m0 = jnp.min(dm, axis=0, keepdims=True)
    rmin_old = jnp.where(i == 0, jnp.float32(3e38), rmin[...])
    better = m0 < rmin_old
    rmin[...] = jnp.where(better, m0, rmin_old)
    ohcT = jnp.where(dm == m0, 1.0, 0.0)
    gc = jnp.dot(snm8_ref[...], ohcT, preferred_element_type=jnp.float32)
    rpay[...] = jnp.where(better, gc, rpay[...])

    @pl.when(i == NBLK - 1)
    def _():
        lossy = jnp.sum(rmin[...]) / P
        a0 = anmt_ref[0:1, :]
        a1 = anmt_ref[1:2, :]
        a2 = anmt_ref[2:3, :]
        an2 = a0 * a0 + a1 * a1 + a2 * a2
        pay = rpay[...]
        p0 = pay[0:1, :]
        p1 = pay[1:2, :]
        p2s = pay[2:3, :]
        sn2_sel = p0 * p0 + p1 * p1 + p2s * p2s
        ndot_c = a0 * p0 + a1 * p1 + a2 * p2s
        cosyx = ndot_c / (jnp.sqrt(an2) * jnp.sqrt(sn2_sel) + 1e-8)
        lny = jnp.sum(1.0 - jnp.abs(cosyx)) / P
        denom = jnp.maximum(acc[1], 1.0)
        out_ref[0, 0] = acc[0] / denom + lossy
        out_ref[0, 1] = acc[2] / denom + lny


def _pad_rows(x, n):
    return jnp.concatenate(
        [x, jnp.zeros((n - x.shape[0], x.shape[1]), x.dtype)], axis=0)


@jax.jit
def kernel(sample_pcd, sample_norm, echo_pcd, aligned_pcd_norm, pcd_vector,
           verts, static_verts):
    sp = _pad_rows(sample_pcd, SPAD)
    snm = _pad_rows(sample_norm, SPAD)
    snm8 = _pad_rows(snm.T, 8)
    ept = echo_pcd.T
    anmt = aligned_pcd_norm.T
    acat = jnp.concatenate(
        [aligned_pcd_norm, jnp.zeros((P, 5), jnp.float32)], axis=1)
    vt = jnp.concatenate(
        [verts.T, jnp.zeros((3, NPAD_TOT - N), verts.dtype)], axis=1)
    st = jnp.concatenate(
        [static_verts.T, jnp.zeros((3, NPAD_TOT - N), verts.dtype)], axis=1)
    vr = vt.reshape(3, NROW, 128)
    sr = st.reshape(3, NROW, 128)

    full = lambda i: (0, 0)
    out = pl.pallas_call(
        _body,
        grid=(NBLK,),
        in_specs=[
            pl.BlockSpec((SBLK, 3), lambda i: (i, 0)),
            pl.BlockSpec((SBLK, 3), lambda i: (i, 0)),
            pl.BlockSpec((8, SBLK), lambda i: (0, i)),
            pl.BlockSpec((3, P), full),
            pl.BlockSpec((3, P), full),
            pl.BlockSpec((P, 8), full),
            pl.BlockSpec(memory_space=pltpu.SMEM),
            pl.BlockSpec((NROW, 128), full),
            pl.BlockSpec((NROW, 128), full),
            pl.BlockSpec((NROW, 128), full),
            pl.BlockSpec((NROW, 128), full),
            pl.BlockSpec((NROW, 128), full),
            pl.BlockSpec((NROW, 128), full),
        ],
        out_specs=pl.BlockSpec(memory_space=pltpu.SMEM),
        out_shape=jax.ShapeDtypeStruct((1, 4), jnp.float32),
        scratch_shapes=[
            pltpu.VMEM((1, P), jnp.float32),
            pltpu.VMEM((8, P), jnp.float32),
            pltpu.VMEM((8, P), jnp.float32),
            pltpu.SMEM((8,), jnp.float32),
        ],
        compiler_params=pltpu.CompilerParams(
            dimension_semantics=("arbitrary",)),
    )(sp, snm, snm8, ept, anmt, acat, pcd_vector,
      vr[0], vr[1], vr[2], sr[0], sr[1], sr[2])

    return out[0]


# speedup vs baseline: 1.4795x; 1.0131x over previous
import jax
import jax.numpy as jnp
from jax import lax
from jax.experimental import pallas as pl
from jax.experimental.pallas import tpu as pltpu

S = 5000
P = 4096
N = 50000
SBLK = 1280
SPAD = 5120
NBLK = SPAD // SBLK
NROW = 392
NPAD_TOT = NROW * 128
NPAD = NPAD_TOT - N
KSEL = 2500
INT_MIN = -2147483648
INT_MAX = 2147483647


def _regs_compute(vx, vy, vz, sx, sy, sz, v0, v1, v2):
    dx = vx - sx
    dy = vy - sy
    dz = vz - sz
    d = dx * v0 + dy * v1 + dz * v2
    axis_reg = jnp.sum(d * d)

    ps = sx * v0 + sy * v1 + sz * v2
    b = lax.bitcast_convert_type(ps, jnp.int32)
    keys = jnp.where(b >= 0, b, jnp.int32(INT_MIN) - b)
    gi = (lax.broadcasted_iota(jnp.int32, (NROW, 128), 0) * 128
          + lax.broadcasted_iota(jnp.int32, (NROW, 128), 1))
    keys = jnp.where(gi < N, keys, jnp.int32(INT_MAX))
    keys2 = ~keys
    k_lo = jnp.int32(KSEL)
    k_hi = jnp.int32(KSEL + NPAD)

    def vstep(t, pq):
        p, q = pq
        shift = jnp.int32(1) << (31 - t)
        candp = p + shift
        candq = q + shift
        cp = jnp.sum(jnp.where(keys < candp, jnp.int32(1), jnp.int32(0)))
        cq = jnp.sum(jnp.where(keys2 < candq, jnp.int32(1), jnp.int32(0)))
        return (jnp.where(cp < k_lo, candp, p),
                jnp.where(cq < k_hi, candq, q))
    t_lo, t2 = lax.fori_loop(
        0, 32, vstep, (jnp.int32(INT_MIN), jnp.int32(INT_MIN)))

    n_lo = k_lo - jnp.sum(jnp.where(keys < t_lo, jnp.int32(1), jnp.int32(0)))
    n_hi = k_hi - jnp.sum(jnp.where(keys2 < t2, jnp.int32(1), jnp.int32(0)))
    ties_lo = keys == t_lo
    ties_hi = keys2 == t2
    rix = jnp.int32(NPAD_TOT - 1) - gi

    def istep(t, pq):
        p, q = pq
        shift = jnp.int32(1) << (16 - t)
        candp = p + shift
        candq = q + shift
        cp = jnp.sum(jnp.where(ties_lo & (gi < candp),
                               jnp.int32(1), jnp.int32(0)))
        cq = jnp.sum(jnp.where(ties_hi & (rix < candq),
                               jnp.int32(1), jnp.int32(0)))
        return (jnp.where(cp < n_lo, candp, p),
                jnp.where(cq < n_hi, candq, q))
    m_lo, m_hi = lax.fori_loop(0, 17, istep, (jnp.int32(0), jnp.int32(0)))

    mask_lo = (keys < t_lo) | (ties_lo & (gi <= m_lo))
    mask_hi = (keys2 < t2) | (ties_hi & (rix <= m_hi))

    d2v = dx * dx + dy * dy + dz * dz
    br = jnp.sum(d2v * (mask_lo.astype(jnp.float32)
                        + mask_hi.astype(jnp.float32)))
    return axis_reg, br


def _body(sp_ref, snm_ref, snm8_ref, ept_ref, anmt_ref, acat_ref, v_ref,
          vx_ref, vy_ref, vz_ref, sx_ref, sy_ref, sz_ref,
          out_ref, rmin, rpay, raug, acc):
    i = pl.program_id(0)
    v0 = v_ref[0, 0]
    v1 = v_ref[0, 1]
    v2 = v_ref[0, 2]

    e0 = ept_ref[0:1, :]
    e1 = ept_ref[1:2, :]
    e2 = ept_ref[2:3, :]
    c0 = jnp.mean(e0)
    c1 = jnp.mean(e1)
    c2 = jnp.mean(e2)
    p2 = (e0 - c0) * v0 + (e1 - c1) * v1 + (e2 - c2) * v2
    mn = jnp.min(p2)
    mx = jnp.max(p2)

    @pl.when(i == 0)
    def _():
        en = e0 * e0 + e1 * e1 + e2 * e2
        raug[...] = jnp.concatenate(
            [-2.0 * e0, -2.0 * e1, -2.0 * e2,
             jnp.ones((1, P), jnp.float32), en,
             jnp.ones((1, P), jnp.float32),
             jnp.zeros((2, P), jnp.float32)], axis=0)
        axis_reg, br = _regs_compute(
            vx_ref[...], vy_ref[...], vz_ref[...],
            sx_ref[...], sy_ref[...], sz_ref[...], v0, v1, v2)
        out_ref[0, 2] = axis_reg
        out_ref[0, 3] = br

    x0 = sp_ref[:, 0:1]
    x1 = sp_ref[:, 1:2]
    x2 = sp_ref[:, 2:3]
    proj1 = (x0 - c0) * v0 + (x1 - c1) * v1 + (x2 - c2) * v2
    rows = i * SBLK + lax.broadcasted_iota(jnp.int32, (SBLK, 1), 0)
    valid = rows < S
    maskf = ((proj1 >= mn) & (proj1 <= mx) & valid).astype(jnp.float32)
    pen = (1.0 - maskf) * 1e10 + jnp.where(valid, 0.0, 1e20)

    sn = x0 * x0 + x1 * x1 + x2 * x2
    lhs = jnp.concatenate(
        [x0, x1, x2, sn, jnp.ones((SBLK, 1), jnp.float32), pen,
         jnp.zeros((SBLK, 2), jnp.float32)], axis=1)
    dm = jnp.maximum(
        jnp.dot(lhs, raug[...], preferred_element_type=jnp.float32), 0.0)

    minx = jnp.min(dm, axis=1, keepdims=True)
    ohr = (dm == minx).astype(jnp.float32)
    g = jnp.dot(ohr, acat_ref[...], preferred_element_type=jnp.float32)
    g0 = g[:, 0:1]
    g1 = g[:, 1:2]
    g2 = g[:, 2:3]
    n0 = snm_ref[:, 0:1]
    n1 = snm_ref[:, 1:2]
    n2 = snm_ref[:, 2:3]
    sn2 = n0 * n0 + n1 * n1 + n2 * n2
    an2_sel = g0 * g0 + g1 * g1 + g2 * g2
    ndot_r = n0 * g0 + n1 * g1 + n2 * g2
    cosxy = ndot_r / (jnp.sqrt(sn2) * jnp.sqrt(an2_sel) + 1e-8)

    s_lossx = jnp.sum(minx * maskf)
    s_denom = jnp.sum(maskf)
    s_lnx = jnp.sum((1.0 - jnp.abs(cosxy)) * maskf)

    @pl.when(i == 0)
    def _():
        acc[0] = s_lossx
        acc[1] = s_denom
        acc[2] = s_lnx

    @pl.when(i > 0)
    def _():
        acc[0] = acc[0] + s_lossx
        acc[1] = acc[1] + s_denom
        acc[2] = acc[2] + s_lnx

    m0 = jnp.min(dm, axis=0, keepdims=True)
    rmin_old = jnp.where(i == 0, jnp.float32(3e38), rmin[...])
    better = m0 < rmin_old
    rmin[...] = jnp.where(better, m0, rmin_old)
    ohcT = jnp.where(dm == m0, 1.0, 0.0)
    gc = jnp.dot(snm8_ref[...], ohcT, preferred_element_type=jnp.float32)
    rpay[...] = jnp.where(better, gc, rpay[...])

    @pl.when(i == NBLK - 1)
    def _():
        lossy = jnp.sum(rmin[...]) / P
        a0 = anmt_ref[0:1, :]
        a1 = anmt_ref[1:2, :]
        a2 = anmt_ref[2:3, :]
        an2 = a0 * a0 + a1 * a1 + a2 * a2
        pay = rpay[...]
        p0 = pay[0:1, :]
        p1 = pay[1:2, :]
        p2s = pay[2:3, :]
        sn2_sel = p0 * p0 + p1 * p1 + p2s * p2s
        ndot_c = a0 * p0 + a1 * p1 + a2 * p2s
        cosyx = ndot_c / (jnp.sqrt(an2) * jnp.sqrt(sn2_sel) + 1e-8)
        lny = jnp.sum(1.0 - jnp.abs(cosyx)) / P
        denom = jnp.maximum(acc[1], 1.0)
        out_ref[0, 0] = acc[0] / denom + lossy
        out_ref[0, 1] = acc[2] / denom + lny


def _pad_rows(x, n):
    return jnp.concatenate(
        [x, jnp.zeros((n - x.shape[0], x.shape[1]), x.dtype)], axis=0)


@jax.jit
def kernel(sample_pcd, sample_norm, echo_pcd, aligned_pcd_norm, pcd_vector,
           verts, static_verts):
    sp = _pad_rows(sample_pcd, SPAD)
    snm = _pad_rows(sample_norm, SPAD)
    snm8 = _pad_rows(snm.T, 8)
    ept = echo_pcd.T
    anmt = aligned_pcd_norm.T
    acat = jnp.concatenate(
        [aligned_pcd_norm, jnp.zeros((P, 5), jnp.float32)], axis=1)
    vt = jnp.concatenate(
        [verts.T, jnp.zeros((3, NPAD_TOT - N), verts.dtype)], axis=1)
    st = jnp.concatenate(
        [static_verts.T, jnp.zeros((3, NPAD_TOT - N), verts.dtype)], axis=1)
    vr = vt.reshape(3, NROW, 128)
    sr = st.reshape(3, NROW, 128)

    full = lambda i: (0, 0)
    out = pl.pallas_call(
        _body,
        grid=(NBLK,),
        in_specs=[
            pl.BlockSpec((SBLK, 3), lambda i: (i, 0)),
            pl.BlockSpec((SBLK, 3), lambda i: (i, 0)),
            pl.BlockSpec((8, SBLK), lambda i: (0, i)),
            pl.BlockSpec((3, P), full),
            pl.BlockSpec((3, P), full),
            pl.BlockSpec((P, 8), full),
            pl.BlockSpec(memory_space=pltpu.SMEM),
            pl.BlockSpec((NROW, 128), full),
            pl.BlockSpec((NROW, 128), full),
            pl.BlockSpec((NROW, 128), full),
            pl.BlockSpec((NROW, 128), full),
            pl.BlockSpec((NROW, 128), full),
            pl.BlockSpec((NROW, 128), full),
        ],
        out_specs=pl.BlockSpec(memory_space=pltpu.SMEM),
        out_shape=jax.ShapeDtypeStruct((1, 4), jnp.float32),
        scratch_shapes=[
            pltpu.VMEM((1, P), jnp.float32),
            pltpu.VMEM((8, P), jnp.float32),
            pltpu.VMEM((8, P), jnp.float32),
            pltpu.SMEM((8,), jnp.float32),
        ],
        compiler_params=pltpu.CompilerParams(
            dimension_semantics=("arbitrary",)),
    )(sp, snm, snm8, ept, anmt, acat, pcd_vector,
      vr[0], vr[1], vr[2], sr[0], sr[1], sr[2])

    return out[0]
